# Initial kernel scaffold; baseline (speedup 1.0000x reference)
import jax
import jax.numpy as jnp
from jax.experimental import pallas as pl


def kernel(x, token_table, pos_table):
    raise NotImplementedError("write your pallas kernel here")


# speedup vs baseline: 2.1155x; 2.1155x over previous
import functools

import jax
import jax.numpy as jnp
from jax import lax
from jax.experimental import pallas as pl
from jax.experimental.pallas import tpu as pltpu
from jax.experimental.pallas import tpu_sc as plsc

VOCAB = 100000
EMBED = 128
MAXLEN = 512
BATCH = 1024

NC = 2
NS = 16
LANES = 16
NW = NC * NS

N_ROWS = BATCH * MAXLEN
ROWS_PER_W = N_ROWS // NW
CHUNK = 128
CHUNKS_PER_W = ROWS_PER_W // CHUNK
POS_PERIOD = MAXLEN // CHUNK


def _body(x_hbm, tok_hbm, pos_hbm, out_hbm, idx_v, rows_v, pos_v, sem):
    wid = lax.axis_index("s") * NC + lax.axis_index("c")
    base = wid * ROWS_PER_W

    pltpu.sync_copy(pos_hbm, pos_v)

    def chunk_body(c, carry):
        start = base + c * CHUNK
        pltpu.sync_copy(x_hbm.at[pl.ds(start, CHUNK)], idx_v)
        pltpu.async_copy(tok_hbm.at[idx_v], rows_v, sem).wait()
        m0 = lax.rem(c, POS_PERIOD) * CHUNK

        def row_body(i, carry2):
            for j in range(EMBED // LANES):
                sl = pl.ds(j * LANES, LANES)
                plsc.addupdate(rows_v.at[i, sl], pos_v[m0 + i, sl])
            return carry2

        lax.fori_loop(0, CHUNK, row_body, 0)
        pltpu.sync_copy(rows_v, out_hbm.at[pl.ds(start, CHUNK)])
        return carry

    lax.fori_loop(0, CHUNKS_PER_W, chunk_body, 0)


@jax.jit
def _embed(idx, token_table, pos_table):
    mesh = plsc.VectorSubcoreMesh(core_axis_name="c", subcore_axis_name="s")
    return pl.kernel(
        _body,
        out_type=jax.ShapeDtypeStruct((N_ROWS, EMBED), jnp.float32),
        mesh=mesh,
        scratch_types=[
            pltpu.VMEM((CHUNK,), jnp.int32),
            pltpu.VMEM((CHUNK, EMBED), jnp.float32),
            pltpu.VMEM((MAXLEN, EMBED), jnp.float32),
            pltpu.SemaphoreType.DMA,
        ],
    )(idx, token_table, pos_table)


def kernel(x, token_table, pos_table):
    b, m = x.shape
    idx = x.reshape(-1).astype(jnp.int32)
    out = _embed(idx, token_table, pos_table)
    return out.reshape(b, m, EMBED)


# speedup vs baseline: 6.7658x; 3.1983x over previous
import jax
import jax.numpy as jnp
from jax import lax
from jax.experimental import pallas as pl
from jax.experimental.pallas import tpu as pltpu
from jax.experimental.pallas import tpu_sc as plsc

VOCAB = 100000
EMBED = 128
MAXLEN = 512
BATCH = 1024

NC = 2
NS = 16
LANES = 16
NW = NC * NS

N_ROWS = BATCH * MAXLEN
ROWS_PER_W = N_ROWS // NW
CHUNK = 64
CHUNKS_PER_W = ROWS_PER_W // CHUNK
POS_PERIOD = MAXLEN // CHUNK
NBUF = 4
AHEAD = 2
ITERS = CHUNKS_PER_W // NBUF


def _body(x_hbm, tok_hbm, pos_hbm, out_hbm,
          idx_v, r0, r1, r2, r3, pos_v,
          g0, g1, g2, g3, s0, s1, s2, s3):
    rows = [r0, r1, r2, r3]
    gsem = [g0, g1, g2, g3]
    ssem = [s0, s1, s2, s3]

    wid = lax.axis_index("s") * NC + lax.axis_index("c")
    cbase = wid * CHUNKS_PER_W
    rbase = wid * ROWS_PER_W

    pltpu.sync_copy(x_hbm.at[pl.ds(cbase, CHUNKS_PER_W)], idx_v)
    pltpu.sync_copy(pos_hbm, pos_v)

    def start_gather(c, b):
        pltpu.async_copy(tok_hbm.at[idx_v.at[c]], rows[b], gsem[b])

    def wait_gather(c, b):
        pltpu.make_async_copy(tok_hbm.at[idx_v.at[c]], rows[b], gsem[b]).wait()

    def start_store(c, b):
        pltpu.async_copy(
            rows[b], out_hbm.at[pl.ds(rbase + c * CHUNK, CHUNK)], ssem[b])

    def wait_store(c, b):
        pltpu.make_async_copy(
            rows[b], out_hbm.at[pl.ds(rbase + c * CHUNK, CHUNK)],
            ssem[b]).wait()

    def add_pos(c, b):
        p0 = lax.rem(c, POS_PERIOD) * CHUNK

        @plsc.parallel_loop(0, CHUNK, unroll=4)
        def _add(r):
            for j in range(EMBED // LANES):
                sl = pl.ds(j * LANES, LANES)
                plsc.addupdate(rows[b].at[r, sl], pos_v[p0 + r, sl])

    for b in range(AHEAD):
        start_gather(b, b)

    for b in range(NBUF):
        c = b
        wait_gather(c, b)
        add_pos(c, b)
        start_store(c, b)
        b2 = (b + AHEAD) % NBUF
        if c + AHEAD >= NBUF:
            wait_store(c + AHEAD - NBUF, b2)
        start_gather(c + AHEAD, b2)

    def iter_body(i, carry):
        c0 = i * NBUF
        for b in range(NBUF):
            c = c0 + b
            wait_gather(c, b)
            add_pos(c, b)
            start_store(c, b)
            b2 = (b + AHEAD) % NBUF

            @pl.when(c + AHEAD < CHUNKS_PER_W)
            def _():
                wait_store(c + AHEAD - NBUF, b2)
                start_gather(c + AHEAD, b2)
        return carry

    lax.fori_loop(1, ITERS, iter_body, 0)

    for b in range(NBUF):
        wait_store(CHUNKS_PER_W - NBUF + b, b)


@jax.jit
def _embed(idx, token_table, pos_table):
    mesh = plsc.VectorSubcoreMesh(core_axis_name="c", subcore_axis_name="s")
    return pl.kernel(
        _body,
        out_type=jax.ShapeDtypeStruct((N_ROWS, EMBED), jnp.float32),
        mesh=mesh,
        scratch_types=[
            pltpu.VMEM((CHUNKS_PER_W, CHUNK), jnp.int32),
        ] + [pltpu.VMEM((CHUNK, EMBED), jnp.float32) for _ in range(NBUF)] + [
            pltpu.VMEM((MAXLEN, EMBED), jnp.float32),
        ] + [pltpu.SemaphoreType.DMA for _ in range(2 * NBUF)],
    )(idx, token_table, pos_table)


def kernel(x, token_table, pos_table):
    b, m = x.shape
    idx = x.reshape(N_ROWS // CHUNK, CHUNK).astype(jnp.int32)
    out = _embed(idx, token_table, pos_table)
    return out.reshape(b, m, EMBED)
